# baseline (speedup 1.0000x reference)
import jax
import jax.numpy as jnp
from jax.experimental import pallas as pl
from jax.experimental.pallas import tpu as pltpu

_LANES = 128
_ROWS_PER_BLOCK = 1024


def _make_fc_body(rpb):
    def _fc_body(xt_ref, wb_ref, o_ref):
        w0 = wb_ref[0, 0]
        w1 = wb_ref[0, 1]
        w2 = wb_ref[0, 2]
        b = wb_ref[0, 3]
        for g in range(rpb // 8):
            s = slice(g * 8 * _LANES, (g + 1) * 8 * _LANES)
            x0 = xt_ref[0, s].reshape(8, _LANES)
            x1 = xt_ref[1, s].reshape(8, _LANES)
            x2 = xt_ref[2, s].reshape(8, _LANES)
            o_ref[g * 8 : (g + 1) * 8, :] = w0 * x0 + w1 * x1 + w2 * x2 + b

    return _fc_body


def kernel(x, weight, bias):
    B, F = x.shape
    assert F == 3

    b_pad = ((B + _LANES - 1) // _LANES) * _LANES
    if b_pad != B:
        x = jnp.pad(x, ((0, b_pad - B), (0, 0)))
    rows = b_pad // _LANES

    xt = x.T

    wb = jnp.concatenate(
        [weight.reshape(F).astype(jnp.float32), bias.astype(jnp.float32)]
    ).reshape(1, 4)

    rpb = min(_ROWS_PER_BLOCK, rows)
    grid = (pl.cdiv(rows, rpb),)

    out = pl.pallas_call(
        _make_fc_body(rpb),
        out_shape=jax.ShapeDtypeStruct((rows, _LANES), jnp.float32),
        grid=grid,
        in_specs=[
            pl.BlockSpec((3, rpb * _LANES), lambda i: (0, i)),
            pl.BlockSpec(memory_space=pltpu.MemorySpace.SMEM),
        ],
        out_specs=pl.BlockSpec((rpb, _LANES), lambda i: (i, 0)),
        compiler_params=pltpu.CompilerParams(
            dimension_semantics=("parallel",),
        ),
        cost_estimate=pl.CostEstimate(
            flops=6 * b_pad, transcendentals=0, bytes_accessed=16 * b_pad),
    )(xt, wb)

    y = out.reshape(b_pad, 1)
    if b_pad != B:
        y = y[:B]
    return y


# speedup vs baseline: 1.2167x; 1.2167x over previous
import jax
import jax.numpy as jnp
from jax.experimental import pallas as pl
from jax.experimental.pallas import tpu as pltpu

_LANES = 128
_ROWS_PER_BLOCK = 4096


def _make_fc_body(rpb):
    half = rpb // 2

    def _emit(xh_ref, o_ref, w0, w1, w2, b, row0):
        for g in range(half // 8):
            s = slice(g * 8 * _LANES, (g + 1) * 8 * _LANES)
            x0 = xh_ref[0, s].reshape(8, _LANES)
            x1 = xh_ref[1, s].reshape(8, _LANES)
            x2 = xh_ref[2, s].reshape(8, _LANES)
            r = row0 + g * 8
            o_ref[r : r + 8, :] = w0 * x0 + w1 * x1 + w2 * x2 + b

    def _fc_body(xa_ref, xb_ref, wb_ref, o_ref):
        w0 = wb_ref[0, 0]
        w1 = wb_ref[0, 1]
        w2 = wb_ref[0, 2]
        b = wb_ref[0, 3]
        _emit(xa_ref, o_ref, w0, w1, w2, b, 0)
        _emit(xb_ref, o_ref, w0, w1, w2, b, half)

    return _fc_body


def kernel(x, weight, bias):
    B, F = x.shape
    assert F == 3

    b_pad = ((B + _LANES - 1) // _LANES) * _LANES
    if b_pad != B:
        x = jnp.pad(x, ((0, b_pad - B), (0, 0)))
    rows = b_pad // _LANES

    xt = x.T

    wb = jnp.concatenate(
        [weight.reshape(F).astype(jnp.float32), bias.astype(jnp.float32)]
    ).reshape(1, 4)

    rpb = min(_ROWS_PER_BLOCK, rows)
    if rpb % 16:
        rpb = max(16, (rpb // 16) * 16)
    grid = (pl.cdiv(rows, rpb),)
    half_lanes = (rpb // 2) * _LANES

    out = pl.pallas_call(
        _make_fc_body(rpb),
        out_shape=jax.ShapeDtypeStruct((rows, _LANES), jnp.float32),
        grid=grid,
        in_specs=[
            pl.BlockSpec((3, half_lanes), lambda i: (0, 2 * i)),
            pl.BlockSpec((3, half_lanes), lambda i: (0, 2 * i + 1)),
            pl.BlockSpec(memory_space=pltpu.MemorySpace.SMEM),
        ],
        out_specs=pl.BlockSpec((rpb, _LANES), lambda i: (i, 0)),
        compiler_params=pltpu.CompilerParams(
            dimension_semantics=("parallel",),
        ),
        cost_estimate=pl.CostEstimate(
            flops=6 * b_pad, transcendentals=0, bytes_accessed=16 * b_pad),
    )(xt, xt, wb)

    y = out.reshape(b_pad, 1)
    if b_pad != B:
        y = y[:B]
    return y


# speedup vs baseline: 1.2174x; 1.0006x over previous
import jax
import jax.numpy as jnp
from jax.experimental import pallas as pl
from jax.experimental.pallas import tpu as pltpu

_LANES = 128
_ROWS_PER_BLOCK = 4096


def _make_fc_body(rpb):
    def _fc_body(xt_ref, wb_ref, o_ref):
        w0 = wb_ref[0, 0]
        w1 = wb_ref[0, 1]
        w2 = wb_ref[0, 2]
        b = wb_ref[0, 3]
        for g in range(rpb // 8):
            s = slice(g * 8 * _LANES, (g + 1) * 8 * _LANES)
            x0 = xt_ref[0, s].reshape(8, _LANES)
            x1 = xt_ref[1, s].reshape(8, _LANES)
            x2 = xt_ref[2, s].reshape(8, _LANES)
            o_ref[g * 8 : (g + 1) * 8, :] = w0 * x0 + w1 * x1 + w2 * x2 + b

    return _fc_body


def kernel(x, weight, bias):
    B, F = x.shape
    assert F == 3

    b_pad = ((B + _LANES - 1) // _LANES) * _LANES
    if b_pad != B:
        x = jnp.pad(x, ((0, b_pad - B), (0, 0)))
    rows = b_pad // _LANES

    xt = x.T

    wb = jnp.concatenate(
        [weight.reshape(F).astype(jnp.float32), bias.astype(jnp.float32)]
    ).reshape(1, 4)

    rpb = min(_ROWS_PER_BLOCK, rows)
    rpb = max(8, (rpb // 8) * 8)
    grid = (pl.cdiv(rows, rpb),)

    out = pl.pallas_call(
        _make_fc_body(rpb),
        out_shape=jax.ShapeDtypeStruct((rows, _LANES), jnp.float32),
        grid=grid,
        in_specs=[
            pl.BlockSpec((3, rpb * _LANES), lambda i: (0, i)),
            pl.BlockSpec(memory_space=pltpu.MemorySpace.SMEM),
        ],
        out_specs=pl.BlockSpec((rpb, _LANES), lambda i: (i, 0)),
        compiler_params=pltpu.CompilerParams(
            dimension_semantics=("parallel",),
        ),
        cost_estimate=pl.CostEstimate(
            flops=6 * b_pad, transcendentals=0, bytes_accessed=16 * b_pad),
    )(xt, wb)

    y = out.reshape(b_pad, 1)
    if b_pad != B:
        y = y[:B]
    return y
